# Initial kernel scaffold; baseline (speedup 1.0000x reference)
import jax
import jax.numpy as jnp
from jax.experimental import pallas as pl


def kernel(title_ids, word_ids, movie_table, word_table):
    raise NotImplementedError("write your pallas kernel here")


# speedup vs baseline: 3.0097x; 3.0097x over previous
import functools

import jax
import jax.numpy as jnp
from jax import lax
from jax.experimental import pallas as pl
from jax.experimental.pallas import tpu as pltpu
from jax.experimental.pallas import tpu_sc as plsc

B = 16384
NFEAT = 32
L = 20
NW = 32
SPW = B // NW
C = 128
NCH = SPW // C
GS = 128
LANES = 16


def _pool_body(tids2d, wids2d, movie, word, out_t, out_w,
               tids_v, trows_v, wids_v, wrows_v, oflat_v, sem):
    wid = lax.axis_index("s") * 2 + lax.axis_index("c")
    base = wid * SPW

    pltpu.sync_copy(tids2d.at[pl.ds((wid // 2) * 8, 8)], tids_v)
    toff = (wid % 2) * (SPW // GS)
    handles = [
        pltpu.async_copy(movie.at[tids_v.at[toff + k]],
                         trows_v.at[pl.ds(k * GS, GS), :], sem)
        for k in range(SPW // GS)
    ]
    for h in handles:
        h.wait()
    pltpu.sync_copy(trows_v, out_t.at[pl.ds(base, SPW)])

    iota = lax.iota(jnp.int32, LANES)
    widrows = SPW * L // 128
    pltpu.sync_copy(wids2d.at[pl.ds(wid * widrows, widrows)], wids_v)
    zerosf = jnp.zeros((LANES,), jnp.float32)

    for c in range(NCH):
        handles = [
            pltpu.async_copy(
                word.at[wids_v.at[c * (C * L // GS) + k]],
                wrows_v.at[pl.ds(k * GS, GS), :], sem)
            for k in range(C * L // GS)
        ]
        for h in handles:
            h.wait()

        def zero_pads(g, carry):
            p = iota + g * LANES
            pf = p + c * (C * L)
            ids = plsc.load_gather(wids_v, [pf >> 7, pf & 127])
            is_pad = ids == 0
            any_pad = jnp.sum(jnp.where(is_pad, 1, 0))
            @pl.when(any_pad > 0)
            def _():
                for f in range(NFEAT):
                    plsc.store_scatter(
                        wrows_v, [p, jnp.full((LANES,), f, jnp.int32)],
                        zerosf, mask=is_pad)
            return carry
        lax.fori_loop(0, C * L // LANES, zero_pads, 0)

        def sample_sum(i, carry):
            h0 = jnp.zeros((LANES,), jnp.float32)
            h1 = jnp.zeros((LANES,), jnp.float32)
            for j in range(L):
                r = i * L + j
                h0 = h0 + wrows_v[r, pl.ds(0, LANES)]
                h1 = h1 + wrows_v[r, pl.ds(LANES, LANES)]
            oflat_v[pl.ds(i * NFEAT, LANES)] = h0
            oflat_v[pl.ds(i * NFEAT + LANES, LANES)] = h1
            return carry
        lax.fori_loop(0, C, sample_sum, 0)

        def block_fix(b, carry):
            cnt = jnp.zeros((LANES,), jnp.float32)
            for j in range(L):
                idx = iota * L + (c * C * L + b * LANES * L + j)
                ids = plsc.load_gather(wids_v, [idx >> 7, idx & 127])
                cnt = cnt + jnp.where(ids != 0, 1.0, 0.0).astype(jnp.float32)
            den = jnp.maximum(cnt, 1.0)
            inv = 1.0 / den
            inv = inv * (2.0 - den * inv)
            inv = inv * (2.0 - den * inv)
            for f in range(NFEAT):
                sidx = iota * NFEAT + (b * LANES * NFEAT + f)
                s = plsc.load_gather(oflat_v, [sidx])
                plsc.store_scatter(oflat_v, [sidx], s * inv)
            return carry
        lax.fori_loop(0, C // LANES, block_fix, 0)

        pltpu.sync_copy(oflat_v,
                        out_w.at[pl.ds((base + c * C) * NFEAT, C * NFEAT)])


def _make_pool():
    return functools.partial(
        pl.kernel,
        out_type=(jax.ShapeDtypeStruct((B, NFEAT), jnp.float32),
                  jax.ShapeDtypeStruct((B * NFEAT,), jnp.float32)),
        mesh=plsc.VectorSubcoreMesh(core_axis_name="c", subcore_axis_name="s"),
        compiler_params=pltpu.CompilerParams(use_tc_tiling_on_sc=False,
                                             needs_layout_passes=False),
        scratch_types=[
            pltpu.VMEM((8, 128), jnp.int32),
            pltpu.VMEM((SPW, NFEAT), jnp.float32),
            pltpu.VMEM((SPW * L // 128, 128), jnp.int32),
            pltpu.VMEM((C * L, NFEAT), jnp.float32),
            pltpu.VMEM((C * NFEAT,), jnp.float32),
            pltpu.SemaphoreType.DMA,
        ],
    )(_pool_body)


def kernel(title_ids, word_ids, movie_table, word_table):
    tids2d = title_ids.reshape(-1, 128)
    wids2d = word_ids.reshape(-1, 128)
    out_t, out_w = _make_pool()(tids2d, wids2d, movie_table, word_table)
    return jnp.concatenate([out_t, out_w.reshape(B, NFEAT)], axis=1)


# speedup vs baseline: 3.0710x; 1.0204x over previous
import functools

import jax
import jax.numpy as jnp
from jax import lax
from jax.experimental import pallas as pl
from jax.experimental.pallas import tpu as pltpu
from jax.experimental.pallas import tpu_sc as plsc

B = 16384
NFEAT = 32
L = 20
NW = 32
SPW = B // NW
C = 128
NCH = SPW // C
GS = 128
LANES = 16


def _pool_body(tids2d, wids2d, movie, word, out,
               tids_v, trows_v, wids_v, wrows_v, oflat_v, sem):
    wid = lax.axis_index("s") * 2 + lax.axis_index("c")
    base = wid * SPW

    pltpu.sync_copy(tids2d.at[pl.ds((wid // 2) * 8, 8)], tids_v)
    toff = (wid % 2) * (SPW // GS)
    handles = [
        pltpu.async_copy(movie.at[tids_v.at[toff + k]],
                         trows_v.at[pl.ds(k * GS, GS), :], sem)
        for k in range(SPW // GS)
    ]
    for h in handles:
        h.wait()
    pltpu.sync_copy(trows_v, out.at[pl.ds(base, SPW), pl.ds(0, NFEAT)])

    iota = lax.iota(jnp.int32, LANES)
    widrows = SPW * L // 128
    pltpu.sync_copy(wids2d.at[pl.ds(wid * widrows, widrows)], wids_v)
    zerosf = jnp.zeros((LANES,), jnp.float32)

    for c in range(NCH):
        handles = [
            pltpu.async_copy(
                word.at[wids_v.at[c * (C * L // GS) + k]],
                wrows_v.at[pl.ds(k * GS, GS), :], sem)
            for k in range(C * L // GS)
        ]
        for h in handles:
            h.wait()

        def zero_pads(g, carry):
            p = iota + g * LANES
            pf = p + c * (C * L)
            ids = plsc.load_gather(wids_v, [pf >> 7, pf & 127])
            is_pad = ids == 0
            any_pad = jnp.sum(jnp.where(is_pad, 1, 0))
            @pl.when(any_pad > 0)
            def _():
                for f in range(NFEAT):
                    plsc.store_scatter(
                        wrows_v, [p, jnp.full((LANES,), f, jnp.int32)],
                        zerosf, mask=is_pad)
            return carry
        lax.fori_loop(0, C * L // LANES, zero_pads, 0)

        def sample_sum(i, carry):
            h0 = jnp.zeros((LANES,), jnp.float32)
            h1 = jnp.zeros((LANES,), jnp.float32)
            for j in range(L):
                r = i * L + j
                h0 = h0 + wrows_v[r, pl.ds(0, LANES)]
                h1 = h1 + wrows_v[r, pl.ds(LANES, LANES)]
            oflat_v[i, pl.ds(0, LANES)] = h0
            oflat_v[i, pl.ds(LANES, LANES)] = h1
            return carry
        lax.fori_loop(0, C, sample_sum, 0)

        def block_fix(b, carry):
            cnt = jnp.zeros((LANES,), jnp.float32)
            for j in range(L):
                idx = iota * L + (c * C * L + b * LANES * L + j)
                ids = plsc.load_gather(wids_v, [idx >> 7, idx & 127])
                cnt = cnt + jnp.where(ids != 0, 1.0, 0.0).astype(jnp.float32)
            den = jnp.maximum(cnt, 1.0)
            inv = 1.0 / den
            inv = inv * (2.0 - den * inv)
            inv = inv * (2.0 - den * inv)
            rows = iota + b * LANES
            for f in range(NFEAT):
                colf = jnp.full((LANES,), f, jnp.int32)
                s = plsc.load_gather(oflat_v, [rows, colf])
                plsc.store_scatter(oflat_v, [rows, colf], s * inv)
            return carry
        lax.fori_loop(0, C // LANES, block_fix, 0)

        pltpu.sync_copy(
            oflat_v, out.at[pl.ds(base + c * C, C), pl.ds(NFEAT, NFEAT)])


def _make_pool():
    return functools.partial(
        pl.kernel,
        out_type=jax.ShapeDtypeStruct((B, 2 * NFEAT), jnp.float32),
        mesh=plsc.VectorSubcoreMesh(core_axis_name="c", subcore_axis_name="s"),
        compiler_params=pltpu.CompilerParams(use_tc_tiling_on_sc=False,
                                             needs_layout_passes=False),
        scratch_types=[
            pltpu.VMEM((8, 128), jnp.int32),
            pltpu.VMEM((SPW, NFEAT), jnp.float32),
            pltpu.VMEM((SPW * L // 128, 128), jnp.int32),
            pltpu.VMEM((C * L, NFEAT), jnp.float32),
            pltpu.VMEM((C, NFEAT), jnp.float32),
            pltpu.SemaphoreType.DMA,
        ],
    )(_pool_body)


def kernel(title_ids, word_ids, movie_table, word_table):
    tids2d = title_ids.reshape(-1, 128)
    wids2d = word_ids.reshape(-1, 128)
    return _make_pool()(tids2d, wids2d, movie_table, word_table)


# speedup vs baseline: 3.3789x; 1.1003x over previous
import functools

import jax
import jax.numpy as jnp
from jax import lax
from jax.experimental import pallas as pl
from jax.experimental.pallas import tpu as pltpu
from jax.experimental.pallas import tpu_sc as plsc

B = 16384
NFEAT = 32
L = 20
NW = 32
SPW = B // NW
C = 128
NCH = SPW // C
GS = 128
LANES = 16

_COMPILER_PARAMS = dict(
    compiler_params=pltpu.CompilerParams(use_tc_tiling_on_sc=False,
                                         needs_layout_passes=False),
)


def _title_body(tids2d, movie, out_t, tids_v, trows_v, sem):
    wid = lax.axis_index("s") * 2 + lax.axis_index("c")
    base = wid * SPW
    pltpu.sync_copy(tids2d.at[pl.ds((wid // 2) * 8, 8)], tids_v)
    toff = (wid % 2) * (SPW // GS)
    handles = [
        pltpu.async_copy(movie.at[tids_v.at[toff + k]],
                         trows_v.at[pl.ds(k * GS, GS), :], sem)
        for k in range(SPW // GS)
    ]
    for h in handles:
        h.wait()
    pltpu.sync_copy(trows_v, out_t.at[pl.ds(base, SPW)])


def _word_body(wids2d, word, out_w, wids_v, wrows_v, oflat_v, sem):
    wid = lax.axis_index("s") * 2 + lax.axis_index("c")
    base = wid * SPW
    iota = lax.iota(jnp.int32, LANES)
    zerosf = jnp.zeros((LANES,), jnp.float32)
    widrows = SPW * L // 128
    pltpu.sync_copy(wids2d.at[pl.ds(wid * widrows, widrows)], wids_v)

    for c in range(NCH):
        handles = [
            pltpu.async_copy(
                word.at[wids_v.at[c * (C * L // GS) + k]],
                wrows_v.at[pl.ds(k * GS, GS), :], sem)
            for k in range(C * L // GS)
        ]
        for h in handles:
            h.wait()

        def zero_pads(g, carry):
            p = iota + g * LANES
            pf = p + c * (C * L)
            ids = plsc.load_gather(wids_v, [pf >> 7, pf & 127])
            is_pad = ids == 0
            any_pad = jnp.sum(jnp.where(is_pad, 1, 0))
            @pl.when(any_pad > 0)
            def _():
                for f in range(NFEAT):
                    plsc.store_scatter(
                        wrows_v, [p, jnp.full((LANES,), f, jnp.int32)],
                        zerosf, mask=is_pad)
            return carry
        lax.fori_loop(0, C * L // LANES, zero_pads, 0)

        def sample_sum(i, carry):
            h0 = jnp.zeros((LANES,), jnp.float32)
            h1 = jnp.zeros((LANES,), jnp.float32)
            for j in range(L):
                r = i * L + j
                h0 = h0 + wrows_v[r, pl.ds(0, LANES)]
                h1 = h1 + wrows_v[r, pl.ds(LANES, LANES)]
            oflat_v[i, pl.ds(0, LANES)] = h0
            oflat_v[i, pl.ds(LANES, LANES)] = h1
            return carry
        lax.fori_loop(0, C, sample_sum, 0)

        def block_fix(b, carry):
            cnt = jnp.zeros((LANES,), jnp.float32)
            for j in range(L):
                idx = iota * L + (c * C * L + b * LANES * L + j)
                ids = plsc.load_gather(wids_v, [idx >> 7, idx & 127])
                cnt = cnt + jnp.where(ids != 0, 1.0, 0.0).astype(jnp.float32)
            den = jnp.maximum(cnt, 1.0)
            inv = 1.0 / den
            inv = inv * (2.0 - den * inv)
            inv = inv * (2.0 - den * inv)
            rows = iota + b * LANES
            for f in range(NFEAT):
                colf = jnp.full((LANES,), f, jnp.int32)
                s = plsc.load_gather(oflat_v, [rows, colf])
                plsc.store_scatter(oflat_v, [rows, colf], s * inv)
            return carry
        lax.fori_loop(0, C // LANES, block_fix, 0)

        pltpu.sync_copy(oflat_v, out_w.at[pl.ds(base + c * C, C)])


def _make_title():
    return functools.partial(
        pl.kernel,
        out_type=jax.ShapeDtypeStruct((B, NFEAT), jnp.float32),
        mesh=plsc.VectorSubcoreMesh(core_axis_name="c", subcore_axis_name="s"),
        scratch_types=[
            pltpu.VMEM((8, 128), jnp.int32),
            pltpu.VMEM((SPW, NFEAT), jnp.float32),
            pltpu.SemaphoreType.DMA,
        ],
        **_COMPILER_PARAMS,
    )(_title_body)


def _make_word():
    return functools.partial(
        pl.kernel,
        out_type=jax.ShapeDtypeStruct((B, NFEAT), jnp.float32),
        mesh=plsc.VectorSubcoreMesh(core_axis_name="c", subcore_axis_name="s"),
        scratch_types=[
            pltpu.VMEM((SPW * L // 128, 128), jnp.int32),
            pltpu.VMEM((C * L, NFEAT), jnp.float32),
            pltpu.VMEM((C, NFEAT), jnp.float32),
            pltpu.SemaphoreType.DMA,
        ],
        **_COMPILER_PARAMS,
    )(_word_body)


def kernel(title_ids, word_ids, movie_table, word_table):
    tids2d = title_ids.reshape(-1, 128)
    wids2d = word_ids.reshape(-1, 128)
    out_w = _make_word()(wids2d, word_table)
    out_t = _make_title()(tids2d, movie_table)
    return jnp.concatenate([out_t, out_w], axis=1)


# speedup vs baseline: 3.3935x; 1.0043x over previous
import functools

import jax
import jax.numpy as jnp
from jax import lax
from jax.experimental import pallas as pl
from jax.experimental.pallas import tpu as pltpu
from jax.experimental.pallas import tpu_sc as plsc

B = 16384
NFEAT = 32
L = 20
NW = 32
SPW = B // NW
NCHW = 8
GS = 128
LANES = 16

_COMPILER_PARAMS = dict(
    compiler_params=pltpu.CompilerParams(use_tc_tiling_on_sc=False,
                                         needs_layout_passes=False),
)


def _title_body(tids2d, movie, out_t, tids_v, trows_v, sem):
    wid = lax.axis_index("s") * 2 + lax.axis_index("c")
    base = wid * SPW
    pltpu.sync_copy(tids2d.at[pl.ds((wid // 2) * 8, 8)], tids_v)
    toff = (wid % 2) * (SPW // GS)
    handles = [
        pltpu.async_copy(movie.at[tids_v.at[toff + k]],
                         trows_v.at[pl.ds(k * GS, GS), :], sem)
        for k in range(SPW // GS)
    ]
    for h in handles:
        h.wait()
    pltpu.sync_copy(trows_v, out_t.at[pl.ds(base, SPW)])


def _word_body(wids2d, word, out_w, wids_v, wrows0_v, wrows1_v, oflat_v,
               sem0, sem1):
    wid = lax.axis_index("s") * 2 + lax.axis_index("c")
    base = wid * SPW
    iota = lax.iota(jnp.int32, LANES)
    zerosf = jnp.zeros((LANES,), jnp.float32)
    widrows = SPW * L // 128
    pltpu.sync_copy(wids2d.at[pl.ds(wid * widrows, widrows)], wids_v)

    CC = SPW // NCHW
    NS = CC * L // GS
    bufs = (wrows0_v, wrows1_v)
    sems = (sem0, sem1)

    def fire(c):
        buf, sem = bufs[c % 2], sems[c % 2]
        return [
            pltpu.async_copy(word.at[wids_v.at[c * NS + k]],
                             buf.at[pl.ds(k * GS, GS), :], sem)
            for k in range(NS)
        ]

    pending = fire(0)
    for c in range(NCHW):
        for h in pending:
            h.wait()
        if c + 1 < NCHW:
            pending = fire(c + 1)
        wrows_v = bufs[c % 2]

        def zero_pads(g, carry):
            p = iota + g * LANES
            pf = p + c * (CC * L)
            ids = plsc.load_gather(wids_v, [pf >> 7, pf & 127])
            is_pad = ids == 0
            any_pad = jnp.sum(jnp.where(is_pad, 1, 0))
            @pl.when(any_pad > 0)
            def _():
                for f in range(NFEAT):
                    plsc.store_scatter(
                        wrows_v, [p, jnp.full((LANES,), f, jnp.int32)],
                        zerosf, mask=is_pad)
            return carry
        lax.fori_loop(0, CC * L // LANES, zero_pads, 0)

        def sample_sum(i, carry):
            h0 = jnp.zeros((LANES,), jnp.float32)
            h1 = jnp.zeros((LANES,), jnp.float32)
            for j in range(L):
                r = i * L + j
                h0 = h0 + wrows_v[r, pl.ds(0, LANES)]
                h1 = h1 + wrows_v[r, pl.ds(LANES, LANES)]
            oflat_v[i, pl.ds(0, LANES)] = h0
            oflat_v[i, pl.ds(LANES, LANES)] = h1
            return carry
        lax.fori_loop(0, CC, sample_sum, 0)

        def block_fix(b, carry):
            cnt = jnp.zeros((LANES,), jnp.float32)
            for j in range(L):
                idx = iota * L + (c * CC * L + b * LANES * L + j)
                ids = plsc.load_gather(wids_v, [idx >> 7, idx & 127])
                cnt = cnt + jnp.where(ids != 0, 1.0, 0.0).astype(jnp.float32)
            den = jnp.maximum(cnt, 1.0)
            inv = 1.0 / den
            inv = inv * (2.0 - den * inv)
            inv = inv * (2.0 - den * inv)
            rows = iota + b * LANES
            for f in range(NFEAT):
                colf = jnp.full((LANES,), f, jnp.int32)
                s = plsc.load_gather(oflat_v, [rows, colf])
                plsc.store_scatter(oflat_v, [rows, colf], s * inv)
            return carry
        lax.fori_loop(0, CC // LANES, block_fix, 0)

        pltpu.sync_copy(oflat_v, out_w.at[pl.ds(base + c * CC, CC)])


def _make_title():
    return functools.partial(
        pl.kernel,
        out_type=jax.ShapeDtypeStruct((B, NFEAT), jnp.float32),
        mesh=plsc.VectorSubcoreMesh(core_axis_name="c", subcore_axis_name="s"),
        scratch_types=[
            pltpu.VMEM((8, 128), jnp.int32),
            pltpu.VMEM((SPW, NFEAT), jnp.float32),
            pltpu.SemaphoreType.DMA,
        ],
        **_COMPILER_PARAMS,
    )(_title_body)


def _make_word():
    return functools.partial(
        pl.kernel,
        out_type=jax.ShapeDtypeStruct((B, NFEAT), jnp.float32),
        mesh=plsc.VectorSubcoreMesh(core_axis_name="c", subcore_axis_name="s"),
        scratch_types=[
            pltpu.VMEM((SPW * L // 128, 128), jnp.int32),
            pltpu.VMEM((SPW // NCHW * L, NFEAT), jnp.float32),
            pltpu.VMEM((SPW // NCHW * L, NFEAT), jnp.float32),
            pltpu.VMEM((SPW // NCHW, NFEAT), jnp.float32),
            pltpu.SemaphoreType.DMA,
            pltpu.SemaphoreType.DMA,
        ],
        **_COMPILER_PARAMS,
    )(_word_body)


def kernel(title_ids, word_ids, movie_table, word_table):
    tids2d = title_ids.reshape(-1, 128)
    wids2d = word_ids.reshape(-1, 128)
    out_w = _make_word()(wids2d, word_table)
    out_t = _make_title()(tids2d, movie_table)
    return jnp.concatenate([out_t, out_w], axis=1)
